# baseline (speedup 1.0000x reference)
import functools

import jax
import jax.numpy as jnp
import numpy as np
from jax.experimental import pallas as pl
from jax.experimental.pallas import tpu as pltpu

_STRIDES = (8, 16, 32, 64, 128)
_IMAGE_SIZE = 512
_NUM_CLASSES = 80
_C = 4 + _NUM_CLASSES
_BATCH = 32
_EPS = 1e-7

_LEVEL_CFG = {
    4096: (8, 1024),
    1024: (8, 1024),
    256: (16, 256),
    64: (32, 64),
    16: (32, 16),
}


def _atan_pos(x):
    big = x > 2.414213562373095
    mid = x > 0.41421356237309503
    y0 = jnp.where(big, np.float32(np.pi / 2),
                   jnp.where(mid, np.float32(np.pi / 4), 0.0))
    xr = jnp.where(big, -1.0 / x, jnp.where(mid, (x - 1.0) / (x + 1.0), x))
    z = xr * xr
    p = (((8.05374449538e-2 * z - 1.38776856032e-1) * z
          + 1.99777106478e-1) * z - 3.33329491539e-1) * z * xr + xr
    return y0 + p


def _block_partials(stride, xb, yb):
    s = np.float32(stride)
    ch = jax.lax.broadcasted_iota(jnp.int32, xb.shape, 2)

    e = jnp.exp(jnp.minimum(xb, -xb))
    ep1 = 1.0 + e
    q = 1.0 / ep1
    p = jnp.where(xb >= 0.0, q, 1.0 - q)
    ce = jnp.maximum(xb, 0.0) - xb * yb + jnp.log(ep1)
    one_m_pt = p + yb - 2.0 * p * yb
    a_t = jnp.where(ch >= 4, 0.75 - 0.5 * yb, 0.0)
    cls_part = jnp.sum(a_t * ce * one_m_pt * one_m_pt)

    my = jnp.where(ch >= 5, yb, -jnp.inf)
    mx = jnp.max(my, axis=2)
    mask = mx != 0.0
    n_part = jnp.sum(mask.astype(jnp.float32))

    dp4 = jnp.swapaxes(jnp.maximum(xb[:, :, 0:4], 0.0), 1, 2) * s
    dt4 = jnp.swapaxes(jnp.maximum(yb[:, :, 0:4], 0.0), 1, 2) * s
    dp0, dp1, dp2, dp3 = dp4[:, 0, :], dp4[:, 1, :], dp4[:, 2, :], dp4[:, 3, :]
    dt0, dt1, dt2, dt3 = dt4[:, 0, :], dt4[:, 1, :], dt4[:, 2, :], dt4[:, 3, :]

    pw = dp0 + dp2
    ph = dp1 + dp3
    tw = dt0 + dt2
    th = dt1 + dt3
    iw = jnp.minimum(dp2, dt2) + jnp.minimum(dp0, dt0)
    ih = jnp.minimum(dp3, dt3) + jnp.minimum(dp1, dt1)
    inter = iw * ih
    union = pw * ph + tw * th - inter
    iou = inter / (union + _EPS)
    cw = jnp.maximum(dp2, dt2) + jnp.maximum(dp0, dt0)
    chh = jnp.maximum(dp3, dt3) + jnp.maximum(dp1, dt1)
    c2 = cw * cw + chh * chh + _EPS
    rho2 = ((dp2 - dp0 - dt2 + dt0) ** 2 + (dp3 - dp1 - dt3 + dt1) ** 2) * 0.25
    v = np.float32(4.0 / np.pi**2) * (
        _atan_pos(tw / (th + _EPS)) - _atan_pos(pw / (ph + _EPS))
    ) ** 2
    alpha = v / (1.0 - iou + v + _EPS)
    ciou = 1.0 - iou + rho2 / c2 + alpha * v
    ciou_part = jnp.sum(jnp.where(mask, ciou, 0.0))
    return n_part, ciou_part, cls_part


def _level_kernel(stride, xa_ref, xb_ref, ya_ref, yb_ref, o_ref):
    na, ca, fa = _block_partials(stride, xa_ref[...], ya_ref[...])
    nb, cb, fb = _block_partials(stride, xb_ref[...], yb_ref[...])

    @pl.when((pl.program_id(0) == 0) & (pl.program_id(1) == 0))
    def _():
        o_ref[0] = 0.0
        o_ref[1] = 0.0
        o_ref[2] = 0.0

    o_ref[0] = o_ref[0] + (na + nb)
    o_ref[1] = o_ref[1] + (ca + cb)
    o_ref[2] = o_ref[2] + (fa + fb)


def _level_partials(l, x, y):
    h = _IMAGE_SIZE // _STRIDES[l]
    hw = h * h
    bb, sb = _LEVEL_CFG[hw]
    x3 = jnp.transpose(x, (0, 2, 3, 1)).reshape(_BATCH, hw, _C)
    y3 = jnp.transpose(y, (0, 2, 3, 1)).reshape(_BATCH, hw, _C)
    sh = sb // 2
    grid = (_BATCH // bb, hw // sb)
    spec_a = pl.BlockSpec((bb, sh, _C), lambda b, w: (b, 2 * w, 0))
    spec_b = pl.BlockSpec((bb, sh, _C), lambda b, w: (b, 2 * w + 1, 0))
    return pl.pallas_call(
        functools.partial(_level_kernel, _STRIDES[l]),
        grid=grid,
        in_specs=[spec_a, spec_b, spec_a, spec_b],
        out_specs=pl.BlockSpec(memory_space=pltpu.SMEM),
        out_shape=jax.ShapeDtypeStruct((3,), jnp.float32),
    )(x3, x3, y3, y3)


def kernel(x_0, x_1, x_2, x_3, x_4, y_0, y_1, y_2, y_3, y_4):
    xs = [x_0, x_1, x_2, x_3, x_4]
    ys = [y_0, y_1, y_2, y_3, y_4]
    parts = [_level_partials(l, xs[l], ys[l]) for l in range(5)]
    p = parts[0] + parts[1] + parts[2] + parts[3] + parts[4]
    n, ciou_sum, cls_sum = p[0], p[1], p[2]
    total_locs = np.float32(
        sum(_BATCH * (_IMAGE_SIZE // s) ** 2 for s in _STRIDES)
    )
    safe_n = jnp.where(n != 0.0, n, 1.0)
    reg_loss = jnp.where(n != 0.0, ciou_sum / safe_n, 0.0)
    cls_loss = jnp.where(n != 0.0, cls_sum / safe_n, cls_sum / total_locs)
    total = reg_loss + cls_loss
    return total, reg_loss, cls_loss


# speedup vs baseline: 1.1283x; 1.1283x over previous
import functools

import jax
import jax.numpy as jnp
import numpy as np
from jax.experimental import pallas as pl
from jax.experimental.pallas import tpu as pltpu

_STRIDES = (8, 16, 32, 64, 128)
_IMAGE_SIZE = 512
_NUM_CLASSES = 80
_C = 4 + _NUM_CLASSES
_BATCH = 32
_EPS = 1e-7

_LEVEL_CFG = {
    4096: (8, 1024),
    1024: (8, 1024),
    256: (16, 256),
    64: (32, 64),
    16: (32, 16),
}


def _atan_pos(x):
    big = x > 2.414213562373095
    mid = x > 0.41421356237309503
    y0 = jnp.where(big, np.float32(np.pi / 2),
                   jnp.where(mid, np.float32(np.pi / 4), 0.0))
    xr = jnp.where(big, -1.0 / x, jnp.where(mid, (x - 1.0) / (x + 1.0), x))
    z = xr * xr
    p = (((8.05374449538e-2 * z - 1.38776856032e-1) * z
          + 1.99777106478e-1) * z - 3.33329491539e-1) * z * xr + xr
    return y0 + p


def _block_partials(stride, xb, yb):
    s = np.float32(stride)
    ch = jax.lax.broadcasted_iota(jnp.int32, xb.shape, 2)

    e = jnp.exp(jnp.minimum(xb, -xb))
    ep1 = 1.0 + e
    q = 1.0 / ep1
    p = jnp.where(xb >= 0.0, q, 1.0 - q)
    ce = jnp.maximum(xb, 0.0) - xb * yb + jnp.log(ep1)
    one_m_pt = p + yb - 2.0 * p * yb
    a_t = jnp.where(ch >= 4, 0.75 - 0.5 * yb, 0.0)
    cls_part = jnp.sum(a_t * ce * one_m_pt * one_m_pt)

    my = jnp.where(ch >= 5, yb, -jnp.inf)
    mx = jnp.max(my, axis=2)
    mask = mx != 0.0
    n_part = jnp.sum(mask.astype(jnp.float32))

    dp4 = jnp.swapaxes(jnp.maximum(xb[:, :, 0:4], 0.0), 1, 2) * s
    dt4 = jnp.swapaxes(jnp.maximum(yb[:, :, 0:4], 0.0), 1, 2) * s
    dp0, dp1, dp2, dp3 = dp4[:, 0, :], dp4[:, 1, :], dp4[:, 2, :], dp4[:, 3, :]
    dt0, dt1, dt2, dt3 = dt4[:, 0, :], dt4[:, 1, :], dt4[:, 2, :], dt4[:, 3, :]

    pw = dp0 + dp2
    ph = dp1 + dp3
    tw = dt0 + dt2
    th = dt1 + dt3
    iw = jnp.minimum(dp2, dt2) + jnp.minimum(dp0, dt0)
    ih = jnp.minimum(dp3, dt3) + jnp.minimum(dp1, dt1)
    inter = iw * ih
    union = pw * ph + tw * th - inter
    iou = inter / (union + _EPS)
    cw = jnp.maximum(dp2, dt2) + jnp.maximum(dp0, dt0)
    chh = jnp.maximum(dp3, dt3) + jnp.maximum(dp1, dt1)
    c2 = cw * cw + chh * chh + _EPS
    rho2 = ((dp2 - dp0 - dt2 + dt0) ** 2 + (dp3 - dp1 - dt3 + dt1) ** 2) * 0.25
    v = np.float32(4.0 / np.pi**2) * (
        _atan_pos(tw / (th + _EPS)) - _atan_pos(pw / (ph + _EPS))
    ) ** 2
    alpha = v / (1.0 - iou + v + _EPS)
    ciou = 1.0 - iou + rho2 / c2 + alpha * v
    ciou_part = jnp.sum(jnp.where(mask, ciou, 0.0))
    return n_part, ciou_part, cls_part


def _fused_kernel(x0, x1, x2, x3, x4, y0, y1, y2, y3, y4, o_ref):
    @pl.when((pl.program_id(0) == 0) & (pl.program_id(1) == 0))
    def _():
        o_ref[0] = 0.0
        o_ref[1] = 0.0
        o_ref[2] = 0.0

    for stride, xr, yr in ((8, x0, y0), (16, x1, y1), (32, x2, y2),
                           (64, x3, y3)):
        n_p, c_p, f_p = _block_partials(stride, xr[...], yr[...])
        o_ref[0] = o_ref[0] + n_p
        o_ref[1] = o_ref[1] + c_p
        o_ref[2] = o_ref[2] + f_p

    @pl.when(pl.program_id(1) == 0)
    def _():
        n_p, c_p, f_p = _block_partials(128, x4[...], y4[...])
        o_ref[0] = o_ref[0] + n_p
        o_ref[1] = o_ref[1] + c_p
        o_ref[2] = o_ref[2] + f_p


def kernel(x_0, x_1, x_2, x_3, x_4, y_0, y_1, y_2, y_3, y_4):
    views = []
    for t in (x_0, x_1, x_2, x_3, x_4, y_0, y_1, y_2, y_3, y_4):
        hw = t.shape[2] * t.shape[3]
        views.append(jnp.transpose(t, (0, 2, 3, 1)).reshape(_BATCH, hw, _C))
    sbs = (1024, 256, 64, 16, 16)
    specs = []
    for i in range(10):
        sb = sbs[i % 5]
        if i % 5 == 4:
            specs.append(pl.BlockSpec((8, sb, _C), lambda b, w: (b, 0, 0)))
        else:
            specs.append(pl.BlockSpec((8, sb, _C), lambda b, w: (b, w, 0)))
    p = pl.pallas_call(
        _fused_kernel,
        grid=(4, 4),
        in_specs=specs,
        out_specs=pl.BlockSpec(memory_space=pltpu.SMEM),
        out_shape=jax.ShapeDtypeStruct((3,), jnp.float32),
    )(*views)
    n, ciou_sum, cls_sum = p[0], p[1], p[2]
    total_locs = np.float32(
        sum(_BATCH * (_IMAGE_SIZE // s) ** 2 for s in _STRIDES)
    )
    safe_n = jnp.where(n != 0.0, n, 1.0)
    reg_loss = jnp.where(n != 0.0, ciou_sum / safe_n, 0.0)
    cls_loss = jnp.where(n != 0.0, cls_sum / safe_n, cls_sum / total_locs)
    total = reg_loss + cls_loss
    return total, reg_loss, cls_loss


# speedup vs baseline: 1.1290x; 1.0007x over previous
import jax
import jax.numpy as jnp
import numpy as np
from jax.experimental import pallas as pl
from jax.experimental.pallas import tpu as pltpu

_STRIDES = (8, 16, 32, 64, 128)
_IMAGE_SIZE = 512
_NUM_CLASSES = 80
_C = 4 + _NUM_CLASSES
_BATCH = 32
_EPS = 1e-7

def _atan_pos(x):
    big = x > 2.414213562373095
    mid = x > 0.41421356237309503
    y0 = jnp.where(big, np.float32(np.pi / 2),
                   jnp.where(mid, np.float32(np.pi / 4), 0.0))
    xr = jnp.where(big, -1.0 / x, jnp.where(mid, (x - 1.0) / (x + 1.0), x))
    z = xr * xr
    p = (((8.05374449538e-2 * z - 1.38776856032e-1) * z
          + 1.99777106478e-1) * z - 3.33329491539e-1) * z * xr + xr
    return y0 + p


def _block_partials(stride, xb, yb):
    s = np.float32(stride)
    ch = jax.lax.broadcasted_iota(jnp.int32, xb.shape, 2)

    e = jnp.exp(jnp.minimum(xb, -xb))
    ep1 = 1.0 + e
    q = 1.0 / ep1
    p = jnp.where(xb >= 0.0, q, 1.0 - q)
    ce = jnp.maximum(xb, 0.0) - xb * yb + jnp.log(ep1)
    one_m_pt = p + yb - 2.0 * p * yb
    a_t = jnp.where(ch >= 4, 0.75 - 0.5 * yb, 0.0)
    cls_part = jnp.sum(a_t * ce * one_m_pt * one_m_pt)

    my = jnp.where(ch >= 5, yb, -jnp.inf)
    mx = jnp.max(my, axis=2)
    mask = mx != 0.0
    n_part = jnp.sum(mask.astype(jnp.float32))

    dp4 = jnp.swapaxes(jnp.maximum(xb[:, :, 0:4], 0.0), 1, 2) * s
    dt4 = jnp.swapaxes(jnp.maximum(yb[:, :, 0:4], 0.0), 1, 2) * s
    dp0, dp1, dp2, dp3 = dp4[:, 0, :], dp4[:, 1, :], dp4[:, 2, :], dp4[:, 3, :]
    dt0, dt1, dt2, dt3 = dt4[:, 0, :], dt4[:, 1, :], dt4[:, 2, :], dt4[:, 3, :]

    pw = dp0 + dp2
    ph = dp1 + dp3
    tw = dt0 + dt2
    th = dt1 + dt3
    iw = jnp.minimum(dp2, dt2) + jnp.minimum(dp0, dt0)
    ih = jnp.minimum(dp3, dt3) + jnp.minimum(dp1, dt1)
    inter = iw * ih
    union = pw * ph + tw * th - inter
    iou = inter / (union + _EPS)
    cw = jnp.maximum(dp2, dt2) + jnp.maximum(dp0, dt0)
    chh = jnp.maximum(dp3, dt3) + jnp.maximum(dp1, dt1)
    c2 = cw * cw + chh * chh + _EPS
    rho2 = ((dp2 - dp0 - dt2 + dt0) ** 2 + (dp3 - dp1 - dt3 + dt1) ** 2) * 0.25
    v = np.float32(4.0 / np.pi**2) * (
        _atan_pos(tw / (th + _EPS)) - _atan_pos(pw / (ph + _EPS))
    ) ** 2
    alpha = v / (1.0 - iou + v + _EPS)
    ciou = 1.0 - iou + rho2 / c2 + alpha * v
    ciou_part = jnp.sum(jnp.where(mask, ciou, 0.0))
    return n_part, ciou_part, cls_part


def _fused_kernel(x0, x1, x2, x3, x4, y0, y1, y2, y3, y4, o_ref):
    @pl.when((pl.program_id(0) == 0) & (pl.program_id(1) == 0))
    def _():
        o_ref[0] = 0.0
        o_ref[1] = 0.0
        o_ref[2] = 0.0

    for stride, xr, yr in ((8, x0, y0), (16, x1, y1), (32, x2, y2),
                           (64, x3, y3)):
        n_p, c_p, f_p = _block_partials(stride, xr[...], yr[...])
        o_ref[0] = o_ref[0] + n_p
        o_ref[1] = o_ref[1] + c_p
        o_ref[2] = o_ref[2] + f_p

    @pl.when(pl.program_id(1) == 0)
    def _():
        n_p, c_p, f_p = _block_partials(128, x4[...], y4[...])
        o_ref[0] = o_ref[0] + n_p
        o_ref[1] = o_ref[1] + c_p
        o_ref[2] = o_ref[2] + f_p


def kernel(x_0, x_1, x_2, x_3, x_4, y_0, y_1, y_2, y_3, y_4):
    views = []
    for t in (x_0, x_1, x_2, x_3, x_4, y_0, y_1, y_2, y_3, y_4):
        hw = t.shape[2] * t.shape[3]
        views.append(jnp.transpose(t, (0, 2, 3, 1)).reshape(_BATCH, hw, _C))
    sbs = (1024, 256, 64, 16, 16)
    specs = []
    for i in range(10):
        sb = sbs[i % 5]
        if i % 5 == 4:
            specs.append(pl.BlockSpec((8, sb, _C), lambda b, w: (b, 0, 0)))
        else:
            specs.append(pl.BlockSpec((8, sb, _C), lambda b, w: (b, w, 0)))
    p = pl.pallas_call(
        _fused_kernel,
        grid=(4, 4),
        in_specs=specs,
        out_specs=pl.BlockSpec(memory_space=pltpu.SMEM),
        out_shape=jax.ShapeDtypeStruct((3,), jnp.float32),
    )(*views)
    n, ciou_sum, cls_sum = p[0], p[1], p[2]
    total_locs = np.float32(
        sum(_BATCH * (_IMAGE_SIZE // s) ** 2 for s in _STRIDES)
    )
    safe_n = jnp.where(n != 0.0, n, 1.0)
    reg_loss = jnp.where(n != 0.0, ciou_sum / safe_n, 0.0)
    cls_loss = jnp.where(n != 0.0, cls_sum / safe_n, cls_sum / total_locs)
    total = reg_loss + cls_loss
    return total, reg_loss, cls_loss
